# Initial kernel scaffold; baseline (speedup 1.0000x reference)
import jax
import jax.numpy as jnp
from jax.experimental import pallas as pl


def kernel(nodes, edges, globals_, senders, receivers, We1, be1, We2, be2, Wn1, bn1, Wn2, bn2, Wg1, bg1, Wg2, bg2):
    raise NotImplementedError("write your pallas kernel here")


# speedup vs baseline: 4.7378x; 4.7378x over previous
import functools

import jax
import jax.numpy as jnp
from jax import lax
from jax.experimental import pallas as pl
from jax.experimental.pallas import tpu as pltpu
from jax.experimental.pallas import tpu_sc as plsc

N_NODES = 10000
N_EDGES = 320000
D_NODE = 128
D_EDGE = 16
HIDDEN = 32

NC = 2
NS = 16
NW = NC * NS
PER_W = N_EDGES // NW
CH = 128
NFULL = PER_W // CH
TAIL = PER_W - NFULL * CH
ROWS_PER_TILE = N_NODES // NS

PACK = 4
ER = N_EDGES // PACK
EBLK = 2000
NEBLK = ER // EBLK

_slope = 0.01


def _leaky(x):
    return jnp.where(x >= 0, x, _slope * x)


def _prep_body(nodes_ref, wr_ref, ws_ref, pr_ref, ps_ref):
    n = nodes_ref[...]
    pr_ref[...] = jnp.dot(n, wr_ref[...], preferred_element_type=jnp.float32)
    ps_ref[...] = jnp.dot(n, ws_ref[...], preferred_element_type=jnp.float32)


def _edge_body(e_ref, g1_ref, g2_ref, w1_ref, b1_ref, w2_ref, b2_ref,
               out_ref, ps_ref):
    i = pl.program_id(0)
    x = jnp.dot(e_ref[...], w1_ref[...], preferred_element_type=jnp.float32)
    x = _leaky(x + b1_ref[...] + g1_ref[...] + g2_ref[...])
    y = jnp.dot(x, w2_ref[...], preferred_element_type=jnp.float32)
    y = _leaky(y + b2_ref[...])
    out_ref[...] = y

    @pl.when(i == 0)
    def _():
        ps_ref[...] = jnp.zeros_like(ps_ref)

    ps_ref[...] += jnp.sum(y, axis=0, keepdims=True)


def _node_body(nodes_ref, agg2_ref, psum_ref,
               wn1a_ref, wn1b_ref, bn1_ref, wn2_ref, bn2_ref,
               wg1_ref, bg1_ref, wg2_ref, bg2_ref,
               nn_ref, g_ref):
    agg = agg2_ref[0] + agg2_ref[1]
    h = jnp.dot(nodes_ref[...], wn1a_ref[...], preferred_element_type=jnp.float32)
    h = h + jnp.dot(agg, wn1b_ref[...], preferred_element_type=jnp.float32)
    h = _leaky(h + bn1_ref[...])
    nn = jnp.dot(h, wn2_ref[...], preferred_element_type=jnp.float32)
    nn = _leaky(nn + bn2_ref[...])
    nn_ref[...] = nn

    node_sum = jnp.sum(nn, axis=0, keepdims=True)
    p = psum_ref[...]
    edge_sum = (p[:, 0:32] + p[:, 32:64] + p[:, 64:96] + p[:, 96:128])
    gi = jnp.dot(node_sum, wg1_ref[0:32, :], preferred_element_type=jnp.float32)
    gi = gi + jnp.dot(edge_sum, wg1_ref[32:64, :], preferred_element_type=jnp.float32)
    gi = _leaky(gi + bg1_ref[...])
    go = jnp.dot(gi, wg2_ref[...], preferred_element_type=jnp.float32)
    g_ref[...] = _leaky(go + bg2_ref[...])


def _sc_gather_body(pr_hbm, ps_hbm, recv_hbm, send_hbm, g1_hbm, g2_hbm,
                    idx_v, rows_v, idx_t, rows_t, sem):
    c = lax.axis_index("c")
    s = lax.axis_index("s")
    base = (s * NC + c) * PER_W

    for tab, idx_hbm, out_hbm in ((pr_hbm, recv_hbm, g1_hbm),
                                  (ps_hbm, send_hbm, g2_hbm)):
        def chunk(j, _, tab=tab, idx_hbm=idx_hbm, out_hbm=out_hbm):
            off = base + j * CH
            pltpu.sync_copy(idx_hbm.at[pl.ds(off, CH)], idx_v)
            pltpu.async_copy(tab.at[idx_v], rows_v, sem).wait()
            pltpu.sync_copy(rows_v, out_hbm.at[pl.ds(off, CH)])
            return _

        lax.fori_loop(0, NFULL, chunk, 0)
        off = base + NFULL * CH
        pltpu.sync_copy(idx_hbm.at[pl.ds(off, TAIL)], idx_t)
        pltpu.async_copy(tab.at[idx_t], rows_t, sem).wait()
        pltpu.sync_copy(rows_t, out_hbm.at[pl.ds(off, TAIL)])


def _sc_scatter_body(ne_hbm, recv_hbm, out_hbm,
                     shared, zbuf, idx_v, rows_v, idx_t, rows_t, sem):
    c = lax.axis_index("c")
    s = lax.axis_index("s")
    base = (s * NC + c) * PER_W

    def zrow(i, _):
        zbuf[i, pl.ds(0, 16)] = jnp.zeros((16,), jnp.float32)
        zbuf[i, pl.ds(16, 16)] = jnp.zeros((16,), jnp.float32)
        return _

    lax.fori_loop(0, ROWS_PER_TILE, zrow, 0)
    pltpu.sync_copy(zbuf, shared.at[pl.ds(s * ROWS_PER_TILE, ROWS_PER_TILE)])
    plsc.subcore_barrier()

    def chunk(j, _):
        off = base + j * CH
        pltpu.sync_copy(recv_hbm.at[pl.ds(off, CH)], idx_v)
        pltpu.sync_copy(ne_hbm.at[pl.ds(off, CH)], rows_v)
        pltpu.sync_copy(rows_v, shared.at[idx_v], add=True)
        return _

    lax.fori_loop(0, NFULL, chunk, 0)
    off = base + NFULL * CH
    pltpu.sync_copy(recv_hbm.at[pl.ds(off, TAIL)], idx_t)
    pltpu.sync_copy(ne_hbm.at[pl.ds(off, TAIL)], rows_t)
    pltpu.sync_copy(rows_t, shared.at[idx_t], add=True)
    plsc.subcore_barrier()

    pltpu.sync_copy(shared.at[pl.ds(s * ROWS_PER_TILE, ROWS_PER_TILE)], zbuf)
    pltpu.sync_copy(zbuf, out_hbm.at[c, pl.ds(s * ROWS_PER_TILE, ROWS_PER_TILE)])


def kernel(nodes, edges, globals_, senders, receivers,
           We1, be1, We2, be2, Wn1, bn1, Wn2, bn2, Wg1, bg1, Wg2, bg2):
    del globals_
    recv = receivers.astype(jnp.int32)
    send = senders.astype(jnp.int32)

    We1e = We1[:D_EDGE]
    We1r = We1[D_EDGE:D_EDGE + D_NODE]
    We1s = We1[D_EDGE + D_NODE:]
    W1bd = jax.scipy.linalg.block_diag(*([We1e] * PACK))
    W2bd = jax.scipy.linalg.block_diag(*([We2] * PACK))
    b1t = jnp.tile(be1, PACK)[None, :]
    b2t = jnp.tile(be2, PACK)[None, :]

    f32 = jnp.float32

    pr, ps = pl.pallas_call(
        _prep_body,
        out_shape=[jax.ShapeDtypeStruct((N_NODES, HIDDEN), f32),
                   jax.ShapeDtypeStruct((N_NODES, HIDDEN), f32)],
    )(nodes, We1r, We1s)

    mesh = plsc.VectorSubcoreMesh(core_axis_name="c", subcore_axis_name="s",
                                  num_cores=NC, num_subcores=NS)
    gather_k = pl.kernel(
        _sc_gather_body,
        out_type=[jax.ShapeDtypeStruct((N_EDGES, HIDDEN), f32),
                  jax.ShapeDtypeStruct((N_EDGES, HIDDEN), f32)],
        mesh=mesh,
        compiler_params=pltpu.CompilerParams(use_tc_tiling_on_sc=False),
        scratch_types=[
            pltpu.VMEM((CH,), jnp.int32),
            pltpu.VMEM((CH, HIDDEN), f32),
            pltpu.VMEM((TAIL,), jnp.int32),
            pltpu.VMEM((TAIL, HIDDEN), f32),
            pltpu.SemaphoreType.DMA,
        ],
    )
    g1, g2 = gather_k(pr, ps, recv, send)

    edges_r = edges.reshape(ER, PACK * D_EDGE)
    g1r = g1.reshape(ER, PACK * HIDDEN)
    g2r = g2.reshape(ER, PACK * HIDDEN)
    new_edges_r, psum = pl.pallas_call(
        _edge_body,
        grid=(NEBLK,),
        in_specs=[
            pl.BlockSpec((EBLK, PACK * D_EDGE), lambda i: (i, 0)),
            pl.BlockSpec((EBLK, PACK * HIDDEN), lambda i: (i, 0)),
            pl.BlockSpec((EBLK, PACK * HIDDEN), lambda i: (i, 0)),
            pl.BlockSpec((PACK * D_EDGE, PACK * HIDDEN), lambda i: (0, 0)),
            pl.BlockSpec((1, PACK * HIDDEN), lambda i: (0, 0)),
            pl.BlockSpec((PACK * HIDDEN, PACK * HIDDEN), lambda i: (0, 0)),
            pl.BlockSpec((1, PACK * HIDDEN), lambda i: (0, 0)),
        ],
        out_specs=[
            pl.BlockSpec((EBLK, PACK * HIDDEN), lambda i: (i, 0)),
            pl.BlockSpec((1, PACK * HIDDEN), lambda i: (0, 0)),
        ],
        out_shape=[jax.ShapeDtypeStruct((ER, PACK * HIDDEN), f32),
                   jax.ShapeDtypeStruct((1, PACK * HIDDEN), f32)],
    )(edges_r, g1r, g2r, W1bd, b1t, W2bd, b2t)
    new_edges = new_edges_r.reshape(N_EDGES, HIDDEN)

    scatter_k = pl.kernel(
        _sc_scatter_body,
        out_type=jax.ShapeDtypeStruct((NC, N_NODES, HIDDEN), f32),
        mesh=mesh,
        compiler_params=pltpu.CompilerParams(use_tc_tiling_on_sc=False),
        scratch_types=[
            pltpu.VMEM_SHARED((N_NODES, HIDDEN), f32),
            pltpu.VMEM((ROWS_PER_TILE, HIDDEN), f32),
            pltpu.VMEM((CH,), jnp.int32),
            pltpu.VMEM((CH, HIDDEN), f32),
            pltpu.VMEM((TAIL,), jnp.int32),
            pltpu.VMEM((TAIL, HIDDEN), f32),
            pltpu.SemaphoreType.DMA,
        ],
    )
    agg2 = scatter_k(new_edges, recv)

    Wn1a = Wn1[:D_NODE]
    Wn1b = Wn1[D_NODE:]
    new_nodes, new_globals = pl.pallas_call(
        _node_body,
        out_shape=[jax.ShapeDtypeStruct((N_NODES, HIDDEN), f32),
                   jax.ShapeDtypeStruct((1, HIDDEN), f32)],
    )(nodes, agg2, psum, Wn1a, Wn1b, bn1[None, :], Wn2, bn2[None, :],
      Wg1, bg1[None, :], Wg2, bg2[None, :])

    return (new_nodes, new_edges, new_globals)
